# baseline (speedup 1.0000x reference)
import functools

import jax
import jax.numpy as jnp
from jax import lax
from jax.experimental import pallas as pl
from jax.experimental.pallas import tpu as pltpu
from jax.experimental.pallas import tpu_sc as plsc

N_NODES = 10000
N_EDGES = 320000
D = 128
HD = D // 2

NC = 2
NS = 16
EB = 128
EPT_RAW = N_EDGES // NS
NB = -(-EPT_RAW // EB)
EPT = 20480
NB = EPT // EB
NP = 10240
ROWS_PER_TILE = NP // NS
NV = EPT // 16
VPB = NV // NB
CCH = 64
PAD_DST = NP - 1


def _sc_segment_sum(h2, srcp, dstp, zrows, zcnt):
    mesh = plsc.VectorSubcoreMesh(
        core_axis_name="c", subcore_axis_name="s", num_cores=NC, num_subcores=NS
    )

    @functools.partial(
        pl.kernel,
        mesh=mesh,
        compiler_params=pltpu.CompilerParams(
            use_tc_tiling_on_sc=False, needs_layout_passes=False
        ),
        out_type=[
            jax.ShapeDtypeStruct((NC, NP, HD), jnp.float32),
            jax.ShapeDtypeStruct((NP,), jnp.float32),
        ],
        scratch_types=[
            pltpu.VMEM((EPT,), jnp.int32),
            pltpu.VMEM((EPT,), jnp.int32),
            pltpu.VMEM((3, EB, HD), jnp.float32),
            pltpu.VMEM((NP,), jnp.float32),
            pltpu.VMEM((NS * CCH,), jnp.float32),
            pltpu.SemaphoreType.DMA,
            pltpu.SemaphoreType.DMA,
            pltpu.SemaphoreType.DMA,
            pltpu.VMEM_SHARED((NP, HD), jnp.float32),
            pltpu.VMEM_SHARED((NS, NP), jnp.float32),
        ],
    )
    def k(h_hbm, src_hbm, dst_hbm, zrows_hbm, zcnt_hbm,
          acc_out, cnt_out, src_v, dst_v, rows_v, cnt_local, cnt_stage,
          sem_g, sem_s, sem_c, acc_sh, cnt_sh):
        cid = lax.axis_index("c")
        sid = lax.axis_index("s")

        pltpu.sync_copy(src_hbm.at[sid], src_v)
        pltpu.sync_copy(dst_hbm.at[sid], dst_v)

        @pl.when(cid == 0)
        def _():
            pltpu.sync_copy(zcnt_hbm, cnt_local)

        r0 = sid * ROWS_PER_TILE
        pltpu.sync_copy(zrows_hbm, acc_sh.at[pl.ds(r0, ROWS_PER_TILE)])
        plsc.subcore_barrier()

        hc = h_hbm.at[cid]
        ones16 = jnp.ones((16,), jnp.float32)

        pltpu.async_copy(hc.at[src_v.at[pl.ds(0, EB)]], rows_v.at[0], sem_g)

        def body(j, carry):
            a = lax.rem(j, 3)

            @pl.when(j >= 2)
            def _():
                pltpu.make_async_copy(zrows_hbm.at[pl.ds(0, EB)], rows_v.at[a],
                                      sem_s).wait()

            @pl.when(j < NB - 1)
            def _():
                pltpu.async_copy(hc.at[src_v.at[pl.ds((j + 1) * EB, EB)]],
                                 rows_v.at[lax.rem(j + 1, 3)], sem_g)

            @pl.when(cid == 0)
            def _():
                def cbody(i, c2):
                    v = j * VPB + i
                    iv = dst_v[pl.ds(v * 16, 16)]
                    plsc.addupdate_scatter(cnt_local, [iv], ones16)
                    return c2

                lax.fori_loop(0, VPB, cbody, 0)

            pltpu.make_async_copy(hc.at[src_v.at[pl.ds(j * EB, EB)]],
                                  rows_v.at[a], sem_g).wait()
            pltpu.async_copy(rows_v.at[a],
                             acc_sh.at[dst_v.at[pl.ds(j * EB, EB)]], sem_s,
                             add=True)
            return carry

        lax.fori_loop(0, NB, body, 0)

        pltpu.make_async_copy(zrows_hbm.at[pl.ds(0, EB)], rows_v.at[0],
                              sem_s).wait()
        pltpu.make_async_copy(zrows_hbm.at[pl.ds(0, EB)], rows_v.at[1],
                              sem_s).wait()

        @pl.when(cid == 0)
        def _():
            pltpu.sync_copy(cnt_local, cnt_sh.at[sid])

        plsc.subcore_barrier()

        @pl.when(cid == 0)
        def _():
            def chunk_body(ch, c2):
                base = r0 + ch * CCH
                for t in range(NS):
                    pltpu.async_copy(cnt_sh.at[t, pl.ds(base, CCH)],
                                     cnt_stage.at[pl.ds(t * CCH, CCH)], sem_c)
                for t in range(NS):
                    pltpu.make_async_copy(cnt_sh.at[t, pl.ds(base, CCH)],
                                          cnt_stage.at[pl.ds(t * CCH, CCH)],
                                          sem_c).wait()

                def rbody(v, c3):
                    s = cnt_stage[pl.ds(v * 16, 16)]
                    for t in range(1, NS):
                        s = s + cnt_stage[pl.ds(t * CCH + v * 16, 16)]
                    cnt_local[pl.ds(ch * CCH + v * 16, 16)] = s
                    return c3

                lax.fori_loop(0, CCH // 16, rbody, 0)
                return c2

            lax.fori_loop(0, ROWS_PER_TILE // CCH, chunk_body, 0)
            pltpu.sync_copy(cnt_local.at[pl.ds(0, ROWS_PER_TILE)],
                            cnt_out.at[pl.ds(r0, ROWS_PER_TILE)])

        pltpu.sync_copy(acc_sh.at[pl.ds(r0, ROWS_PER_TILE)],
                        acc_out.at[cid, pl.ds(r0, ROWS_PER_TILE)])

    return k(h2, srcp, dstp, zrows, zcnt)


def _tc_apply(h, acc, cnt, wt, b2):
    R = 1000

    def body(h_ref, acc_ref, cnt_ref, wt_ref, b_ref, o_ref):
        hb = h_ref[...]
        deg = jnp.maximum(cnt_ref[...], 1.0)
        c0 = acc_ref[0] / deg
        c1 = acc_ref[1] / deg
        z = (
            jnp.dot(hb, wt_ref[0:D, :], preferred_element_type=jnp.float32)
            + jnp.dot(c0, wt_ref[D:D + HD, :], preferred_element_type=jnp.float32)
            + jnp.dot(c1, wt_ref[D + HD:2 * D, :],
                      preferred_element_type=jnp.float32)
            + b_ref[...]
        )
        n = jnp.sqrt(jnp.sum(z * z, axis=1, keepdims=True))
        z = z / jnp.maximum(n, 1e-12)
        o_ref[...] = hb + jnp.maximum(z, 0.0)

    return pl.pallas_call(
        body,
        grid=(N_NODES // R,),
        in_specs=[
            pl.BlockSpec((R, D), lambda i: (i, 0)),
            pl.BlockSpec((NC, R, HD), lambda i: (0, i, 0)),
            pl.BlockSpec((R, 1), lambda i: (i, 0)),
            pl.BlockSpec((2 * D, D), lambda i: (0, 0)),
            pl.BlockSpec((1, D), lambda i: (0, 0)),
        ],
        out_specs=pl.BlockSpec((R, D), lambda i: (i, 0)),
        out_shape=jax.ShapeDtypeStruct((N_NODES, D), jnp.float32),
    )(h, acc, cnt, wt, b2)


@jax.jit
def kernel(h, edge_index, W, b):
    ei = edge_index.astype(jnp.int32)
    srcp = jnp.pad(ei[0].reshape(NS, EPT_RAW), ((0, 0), (0, EPT - EPT_RAW)))
    dstp = jnp.pad(ei[1].reshape(NS, EPT_RAW), ((0, 0), (0, EPT - EPT_RAW)),
                   constant_values=PAD_DST)
    h2 = h.reshape(N_NODES, NC, HD).transpose(1, 0, 2)
    zrows = jnp.zeros((ROWS_PER_TILE, HD), jnp.float32)
    zcnt = jnp.zeros((NP,), jnp.float32)
    acc, cnt = _sc_segment_sum(h2, srcp, dstp, zrows, zcnt)
    wt = W.T
    b2 = b.reshape(1, D)
    return _tc_apply(h, acc, cnt.reshape(NP, 1), wt, b2)


# speedup vs baseline: 1.0003x; 1.0003x over previous
import functools

import jax
import jax.numpy as jnp
from jax import lax
from jax.experimental import pallas as pl
from jax.experimental.pallas import tpu as pltpu
from jax.experimental.pallas import tpu_sc as plsc

N_NODES = 10000
N_EDGES = 320000
D = 128
HD = D // 2

NC = 2
NS = 16
EB = 128
EPT_RAW = N_EDGES // NS
NB = -(-EPT_RAW // EB)
EPT = 20480
NB = EPT // EB
NP = 10240
ROWS_PER_TILE = NP // NS
NV = EPT // 16
VPB = NV // NB
CCH = 64
PAD_DST = NP - 1


def _sc_segment_sum(h2, srcp, dstp, zrows, zcnt):
    mesh = plsc.VectorSubcoreMesh(
        core_axis_name="c", subcore_axis_name="s", num_cores=NC, num_subcores=NS
    )

    @functools.partial(
        pl.kernel,
        mesh=mesh,
        compiler_params=pltpu.CompilerParams(
            use_tc_tiling_on_sc=False, needs_layout_passes=False
        ),
        out_type=[
            jax.ShapeDtypeStruct((NC, NP, HD), jnp.float32),
            jax.ShapeDtypeStruct((NP,), jnp.float32),
        ],
        scratch_types=[
            pltpu.VMEM((NB, EB), jnp.int32),
            pltpu.VMEM((NB, EB), jnp.int32),
            pltpu.VMEM((3, EB, HD), jnp.float32),
            pltpu.VMEM((NP,), jnp.float32),
            pltpu.VMEM((NS * CCH,), jnp.float32),
            pltpu.SemaphoreType.DMA,
            pltpu.SemaphoreType.DMA,
            pltpu.SemaphoreType.DMA,
            pltpu.VMEM_SHARED((NP, HD), jnp.float32),
            pltpu.VMEM_SHARED((NS, NP), jnp.float32),
        ],
    )
    def k(h_hbm, src_hbm, dst_hbm, zrows_hbm, zcnt_hbm,
          acc_out, cnt_out, src_v, dst_v, rows_v, cnt_local, cnt_stage,
          sem_g, sem_s, sem_c, acc_sh, cnt_sh):
        cid = lax.axis_index("c")
        sid = lax.axis_index("s")

        pltpu.sync_copy(src_hbm.at[sid], src_v)
        pltpu.sync_copy(dst_hbm.at[sid], dst_v)

        @pl.when(cid == 0)
        def _():
            pltpu.sync_copy(zcnt_hbm, cnt_local)

        r0 = sid * ROWS_PER_TILE
        pltpu.sync_copy(zrows_hbm, acc_sh.at[pl.ds(r0, ROWS_PER_TILE)])
        plsc.subcore_barrier()

        hc = h_hbm.at[cid]
        ones16 = jnp.ones((16,), jnp.float32)

        pltpu.async_copy(hc.at[src_v.at[0]], rows_v.at[0], sem_g)

        def body(j, carry):
            a = lax.rem(j, 3)

            @pl.when(j >= 2)
            def _():
                pltpu.make_async_copy(zrows_hbm.at[pl.ds(0, EB)], rows_v.at[a],
                                      sem_s).wait()

            @pl.when(j < NB - 1)
            def _():
                pltpu.async_copy(hc.at[src_v.at[j + 1]],
                                 rows_v.at[lax.rem(j + 1, 3)], sem_g)

            @pl.when(cid == 0)
            def _():
                def cbody(i, c2):
                    iv = dst_v[j, pl.ds(i * 16, 16)]
                    plsc.addupdate_scatter(cnt_local, [iv], ones16)
                    return c2

                lax.fori_loop(0, VPB, cbody, 0)

            pltpu.make_async_copy(hc.at[src_v.at[j]],
                                  rows_v.at[a], sem_g).wait()
            pltpu.async_copy(rows_v.at[a],
                             acc_sh.at[dst_v.at[j]], sem_s,
                             add=True)
            return carry

        lax.fori_loop(0, NB, body, 0)

        pltpu.make_async_copy(zrows_hbm.at[pl.ds(0, EB)], rows_v.at[0],
                              sem_s).wait()
        pltpu.make_async_copy(zrows_hbm.at[pl.ds(0, EB)], rows_v.at[1],
                              sem_s).wait()

        @pl.when(cid == 0)
        def _():
            pltpu.sync_copy(cnt_local, cnt_sh.at[sid])

        plsc.subcore_barrier()

        @pl.when(cid == 0)
        def _():
            def chunk_body(ch, c2):
                base = r0 + ch * CCH
                for t in range(NS):
                    pltpu.async_copy(cnt_sh.at[t, pl.ds(base, CCH)],
                                     cnt_stage.at[pl.ds(t * CCH, CCH)], sem_c)
                for t in range(NS):
                    pltpu.make_async_copy(cnt_sh.at[t, pl.ds(base, CCH)],
                                          cnt_stage.at[pl.ds(t * CCH, CCH)],
                                          sem_c).wait()

                def rbody(v, c3):
                    s = cnt_stage[pl.ds(v * 16, 16)]
                    for t in range(1, NS):
                        s = s + cnt_stage[pl.ds(t * CCH + v * 16, 16)]
                    cnt_local[pl.ds(ch * CCH + v * 16, 16)] = s
                    return c3

                lax.fori_loop(0, CCH // 16, rbody, 0)
                return c2

            lax.fori_loop(0, ROWS_PER_TILE // CCH, chunk_body, 0)
            pltpu.sync_copy(cnt_local.at[pl.ds(0, ROWS_PER_TILE)],
                            cnt_out.at[pl.ds(r0, ROWS_PER_TILE)])

        pltpu.sync_copy(acc_sh.at[pl.ds(r0, ROWS_PER_TILE)],
                        acc_out.at[cid, pl.ds(r0, ROWS_PER_TILE)])

    return k(h2, srcp, dstp, zrows, zcnt)


def _tc_apply(h, acc, cnt, wt, b2):
    R = 1000

    def body(h_ref, acc_ref, cnt_ref, wt_ref, b_ref, o_ref):
        hb = h_ref[...]
        deg = jnp.maximum(cnt_ref[...], 1.0)
        c0 = acc_ref[0] / deg
        c1 = acc_ref[1] / deg
        z = (
            jnp.dot(hb, wt_ref[0:D, :], preferred_element_type=jnp.float32)
            + jnp.dot(c0, wt_ref[D:D + HD, :], preferred_element_type=jnp.float32)
            + jnp.dot(c1, wt_ref[D + HD:2 * D, :],
                      preferred_element_type=jnp.float32)
            + b_ref[...]
        )
        n = jnp.sqrt(jnp.sum(z * z, axis=1, keepdims=True))
        z = z / jnp.maximum(n, 1e-12)
        o_ref[...] = hb + jnp.maximum(z, 0.0)

    return pl.pallas_call(
        body,
        grid=(N_NODES // R,),
        in_specs=[
            pl.BlockSpec((R, D), lambda i: (i, 0)),
            pl.BlockSpec((NC, R, HD), lambda i: (0, i, 0)),
            pl.BlockSpec((R, 1), lambda i: (i, 0)),
            pl.BlockSpec((2 * D, D), lambda i: (0, 0)),
            pl.BlockSpec((1, D), lambda i: (0, 0)),
        ],
        out_specs=pl.BlockSpec((R, D), lambda i: (i, 0)),
        out_shape=jax.ShapeDtypeStruct((N_NODES, D), jnp.float32),
    )(h, acc, cnt, wt, b2)


@jax.jit
def kernel(h, edge_index, W, b):
    ei = edge_index.astype(jnp.int32)
    srcp = jnp.pad(ei[0].reshape(NS, EPT_RAW), ((0, 0), (0, EPT - EPT_RAW))
                   ).reshape(NS, NB, EB)
    dstp = jnp.pad(ei[1].reshape(NS, EPT_RAW), ((0, 0), (0, EPT - EPT_RAW)),
                   constant_values=PAD_DST).reshape(NS, NB, EB)
    h2 = h.reshape(N_NODES, NC, HD).transpose(1, 0, 2)
    zrows = jnp.zeros((ROWS_PER_TILE, HD), jnp.float32)
    zcnt = jnp.zeros((NP,), jnp.float32)
    acc, cnt = _sc_segment_sum(h2, srcp, dstp, zrows, zcnt)
    wt = W.T
    b2 = b.reshape(1, D)
    return _tc_apply(h, acc, cnt.reshape(NP, 1), wt, b2)


# speedup vs baseline: 1.0053x; 1.0050x over previous
import functools

import jax
import jax.numpy as jnp
from jax import lax
from jax.experimental import pallas as pl
from jax.experimental.pallas import tpu as pltpu
from jax.experimental.pallas import tpu_sc as plsc

N_NODES = 10000
N_EDGES = 320000
D = 128
HD = D // 2

NC = 2
NS = 16
EB = 128
EPT_RAW = N_EDGES // NS
NB = -(-EPT_RAW // EB)
EPT = 20480
NB = EPT // EB
NP = 10240
ROWS_PER_TILE = NP // NS
NV = EPT // 16
VPB = NV // NB
CCH = 64
PAD_DST = NP - 1


def _sc_segment_sum(h2, srcp, dstp, zrows, zcnt):
    mesh = plsc.VectorSubcoreMesh(
        core_axis_name="c", subcore_axis_name="s", num_cores=NC, num_subcores=NS
    )

    @functools.partial(
        pl.kernel,
        mesh=mesh,
        compiler_params=pltpu.CompilerParams(
            use_tc_tiling_on_sc=False, needs_layout_passes=False
        ),
        out_type=[
            jax.ShapeDtypeStruct((NC, NP, HD), jnp.float32),
            jax.ShapeDtypeStruct((NP,), jnp.float32),
        ],
        scratch_types=[
            pltpu.VMEM((NB, EB), jnp.int32),
            pltpu.VMEM((NB, EB), jnp.int32),
            pltpu.VMEM((3, EB, HD), jnp.float32),
            pltpu.VMEM((NP,), jnp.float32),
            pltpu.VMEM((NS * CCH,), jnp.float32),
            pltpu.SemaphoreType.DMA,
            pltpu.SemaphoreType.DMA,
            pltpu.SemaphoreType.DMA,
            pltpu.VMEM_SHARED((NP, HD), jnp.float32),
            pltpu.VMEM_SHARED((NS, NP), jnp.float32),
        ],
    )
    def k(h_hbm, src_hbm, dst_hbm, zrows_hbm, zcnt_hbm,
          acc_out, cnt_out, src_v, dst_v, rows_v, cnt_local, cnt_stage,
          sem_g, sem_s, sem_c, acc_sh, cnt_sh):
        cid = lax.axis_index("c")
        sid = lax.axis_index("s")

        pltpu.sync_copy(src_hbm.at[sid], src_v)
        pltpu.sync_copy(dst_hbm.at[sid], dst_v)

        @pl.when(cid == 0)
        def _():
            pltpu.sync_copy(zcnt_hbm, cnt_local)

        r0 = sid * ROWS_PER_TILE
        pltpu.sync_copy(zrows_hbm, acc_sh.at[pl.ds(r0, ROWS_PER_TILE)])
        plsc.subcore_barrier()

        hc = h_hbm.at[cid]
        ones16 = jnp.ones((16,), jnp.float32)

        pltpu.async_copy(hc.at[src_v.at[0]], rows_v.at[0], sem_g)

        def body(j, carry):
            a = lax.rem(j, 3)

            @pl.when(j >= 2)
            def _():
                pltpu.make_async_copy(zrows_hbm.at[pl.ds(0, EB)], rows_v.at[a],
                                      sem_s).wait()

            @pl.when(j < NB - 1)
            def _():
                pltpu.async_copy(hc.at[src_v.at[j + 1]],
                                 rows_v.at[lax.rem(j + 1, 3)], sem_g)

            @pl.when(cid == 0)
            def _():
                def cbody(i, c2):
                    iv = dst_v[j, pl.ds(i * 16, 16)]
                    plsc.addupdate_scatter(cnt_local, [iv], ones16)
                    return c2

                lax.fori_loop(0, VPB, cbody, 0)

            pltpu.make_async_copy(hc.at[src_v.at[j]],
                                  rows_v.at[a], sem_g).wait()
            pltpu.async_copy(rows_v.at[a],
                             acc_sh.at[dst_v.at[j]], sem_s,
                             add=True)
            return carry

        lax.fori_loop(0, NB, body, 0)

        pltpu.make_async_copy(zrows_hbm.at[pl.ds(0, EB)], rows_v.at[0],
                              sem_s).wait()
        pltpu.make_async_copy(zrows_hbm.at[pl.ds(0, EB)], rows_v.at[1],
                              sem_s).wait()

        @pl.when(cid == 0)
        def _():
            pltpu.sync_copy(cnt_local, cnt_sh.at[sid])

        plsc.subcore_barrier()

        @pl.when(cid == 0)
        def _():
            def chunk_body(ch, c2):
                base = r0 + ch * CCH
                for t in range(NS):
                    pltpu.async_copy(cnt_sh.at[t, pl.ds(base, CCH)],
                                     cnt_stage.at[pl.ds(t * CCH, CCH)], sem_c)
                for t in range(NS):
                    pltpu.make_async_copy(cnt_sh.at[t, pl.ds(base, CCH)],
                                          cnt_stage.at[pl.ds(t * CCH, CCH)],
                                          sem_c).wait()

                def rbody(v, c3):
                    s = cnt_stage[pl.ds(v * 16, 16)]
                    for t in range(1, NS):
                        s = s + cnt_stage[pl.ds(t * CCH + v * 16, 16)]
                    cnt_local[pl.ds(ch * CCH + v * 16, 16)] = s
                    return c3

                lax.fori_loop(0, CCH // 16, rbody, 0)
                return c2

            lax.fori_loop(0, ROWS_PER_TILE // CCH, chunk_body, 0)
            pltpu.sync_copy(cnt_local.at[pl.ds(0, ROWS_PER_TILE)],
                            cnt_out.at[pl.ds(r0, ROWS_PER_TILE)])

        pltpu.sync_copy(acc_sh.at[pl.ds(r0, ROWS_PER_TILE)],
                        acc_out.at[cid, pl.ds(r0, ROWS_PER_TILE)])

    return k(h2, srcp, dstp, zrows, zcnt)


def _tc_apply(h, acc, cnt, wt, b2):
    R = 1000

    def body(h_ref, acc_ref, cnt_ref, wt_ref, b_ref, o_ref):
        hb = h_ref[...]
        deg = jnp.maximum(cnt_ref[...], 1.0)
        c0 = acc_ref[0] / deg
        c1 = acc_ref[1] / deg
        z = (
            jnp.dot(hb, wt_ref[0:D, :], preferred_element_type=jnp.float32)
            + jnp.dot(c0, wt_ref[D:D + HD, :], preferred_element_type=jnp.float32)
            + jnp.dot(c1, wt_ref[D + HD:2 * D, :],
                      preferred_element_type=jnp.float32)
            + b_ref[...]
        )
        n = jnp.sqrt(jnp.sum(z * z, axis=1, keepdims=True))
        z = z / jnp.maximum(n, 1e-12)
        o_ref[...] = hb + jnp.maximum(z, 0.0)

    return pl.pallas_call(
        body,
        grid=(N_NODES // R,),
        in_specs=[
            pl.BlockSpec((R, D), lambda i: (i, 0)),
            pl.BlockSpec((NC, R, HD), lambda i: (0, i, 0)),
            pl.BlockSpec((R, 1), lambda i: (i, 0)),
            pl.BlockSpec((2 * D, D), lambda i: (0, 0)),
            pl.BlockSpec((1, D), lambda i: (0, 0)),
        ],
        out_specs=pl.BlockSpec((R, D), lambda i: (i, 0)),
        out_shape=jax.ShapeDtypeStruct((N_NODES, D), jnp.float32),
    )(h, acc, cnt, wt, b2)


@jax.jit
def kernel(h, edge_index, W, b):
    ei = edge_index.astype(jnp.int32)
    srcp = jnp.pad(ei[0].reshape(NS, EPT_RAW), ((0, 0), (0, EPT - EPT_RAW))
                   ).reshape(NS, NB, EB)
    padv = N_NODES + (jnp.arange(EPT - EPT_RAW) % (NP - N_NODES))
    padv = jnp.broadcast_to(padv[None, :], (NS, EPT - EPT_RAW)).astype(jnp.int32)
    dstp = jnp.concatenate([ei[1].reshape(NS, EPT_RAW), padv],
                           axis=1).reshape(NS, NB, EB)
    h2 = h.reshape(N_NODES, NC, HD).transpose(1, 0, 2)
    zrows = jnp.zeros((ROWS_PER_TILE, HD), jnp.float32)
    zcnt = jnp.zeros((NP,), jnp.float32)
    acc, cnt = _sc_segment_sum(h2, srcp, dstp, zrows, zcnt)
    wt = W.T
    b2 = b.reshape(1, D)
    return _tc_apply(h, acc, cnt.reshape(NP, 1), wt, b2)


# speedup vs baseline: 1.6308x; 1.6223x over previous
import functools

import jax
import jax.numpy as jnp
from jax import lax
from jax.experimental import pallas as pl
from jax.experimental.pallas import tpu as pltpu
from jax.experimental.pallas import tpu_sc as plsc

N_NODES = 10000
N_EDGES = 320000
D = 128
HD = D // 2

NC = 2
NS = 16
EB = 112
EPT_RAW = N_EDGES // NS
NB = -(-EPT_RAW // EB)
EPT = NB * EB
NP = 10240
ROWS_PER_TILE = NP // NS
NV = EPT // 16
VPB = NV // NB
CCH = 64
PAD_DST = NP - 1


def _sc_segment_sum(h2, srcp, dstp, zrows, zcnt):
    mesh = plsc.VectorSubcoreMesh(
        core_axis_name="c", subcore_axis_name="s", num_cores=NC, num_subcores=NS
    )

    @functools.partial(
        pl.kernel,
        mesh=mesh,
        compiler_params=pltpu.CompilerParams(
            use_tc_tiling_on_sc=False, needs_layout_passes=False
        ),
        out_type=[
            jax.ShapeDtypeStruct((NC, NP, HD), jnp.float32),
            jax.ShapeDtypeStruct((NP,), jnp.float32),
        ],
        scratch_types=[
            pltpu.VMEM((NB, EB), jnp.int32),
            pltpu.VMEM((NB, EB), jnp.int32),
            pltpu.VMEM((3, EB, HD), jnp.float32),
            pltpu.VMEM((NP,), jnp.float32),
            pltpu.VMEM((NS * CCH,), jnp.float32),
            pltpu.SemaphoreType.DMA,
            pltpu.SemaphoreType.DMA,
            pltpu.SemaphoreType.DMA,
            pltpu.VMEM_SHARED((NP, HD), jnp.float32),
            pltpu.VMEM_SHARED((NS, NP), jnp.float32),
        ],
    )
    def k(h_hbm, src_hbm, dst_hbm, zrows_hbm, zcnt_hbm,
          acc_out, cnt_out, src_v, dst_v, rows_v, cnt_local, cnt_stage,
          sem_g, sem_s, sem_c, acc_sh, cnt_sh):
        cid = lax.axis_index("c")
        sid = lax.axis_index("s")

        pltpu.sync_copy(src_hbm.at[sid], src_v)
        pltpu.sync_copy(dst_hbm.at[sid], dst_v)

        @pl.when(cid == 0)
        def _():
            pltpu.sync_copy(zcnt_hbm, cnt_local)

        r0 = sid * ROWS_PER_TILE
        pltpu.sync_copy(zrows_hbm, acc_sh.at[pl.ds(r0, ROWS_PER_TILE)])
        plsc.subcore_barrier()

        hc = h_hbm.at[cid]
        ones16 = jnp.ones((16,), jnp.float32)

        pltpu.async_copy(hc.at[src_v.at[0]], rows_v.at[0], sem_g)

        def body(j, carry):
            a = lax.rem(j, 3)

            @pl.when(j >= 2)
            def _():
                pltpu.make_async_copy(zrows_hbm.at[pl.ds(0, EB)], rows_v.at[a],
                                      sem_s).wait()

            @pl.when(j < NB - 1)
            def _():
                pltpu.async_copy(hc.at[src_v.at[j + 1]],
                                 rows_v.at[lax.rem(j + 1, 3)], sem_g)

            @pl.when(cid == 0)
            def _():
                def cbody(i, c2):
                    iv = dst_v[j, pl.ds(i * 16, 16)]
                    plsc.addupdate_scatter(cnt_local, [iv], ones16)
                    return c2

                lax.fori_loop(0, VPB, cbody, 0)

            pltpu.make_async_copy(hc.at[src_v.at[j]],
                                  rows_v.at[a], sem_g).wait()
            pltpu.async_copy(rows_v.at[a],
                             acc_sh.at[dst_v.at[j]], sem_s,
                             add=True)
            return carry

        lax.fori_loop(0, NB, body, 0)

        pltpu.make_async_copy(zrows_hbm.at[pl.ds(0, EB)], rows_v.at[0],
                              sem_s).wait()
        pltpu.make_async_copy(zrows_hbm.at[pl.ds(0, EB)], rows_v.at[1],
                              sem_s).wait()

        @pl.when(cid == 0)
        def _():
            pltpu.sync_copy(cnt_local, cnt_sh.at[sid])

        plsc.subcore_barrier()

        @pl.when(cid == 0)
        def _():
            def chunk_body(ch, c2):
                base = r0 + ch * CCH
                for t in range(NS):
                    pltpu.async_copy(cnt_sh.at[t, pl.ds(base, CCH)],
                                     cnt_stage.at[pl.ds(t * CCH, CCH)], sem_c)
                for t in range(NS):
                    pltpu.make_async_copy(cnt_sh.at[t, pl.ds(base, CCH)],
                                          cnt_stage.at[pl.ds(t * CCH, CCH)],
                                          sem_c).wait()

                def rbody(v, c3):
                    s = cnt_stage[pl.ds(v * 16, 16)]
                    for t in range(1, NS):
                        s = s + cnt_stage[pl.ds(t * CCH + v * 16, 16)]
                    cnt_local[pl.ds(ch * CCH + v * 16, 16)] = s
                    return c3

                lax.fori_loop(0, CCH // 16, rbody, 0)
                return c2

            lax.fori_loop(0, ROWS_PER_TILE // CCH, chunk_body, 0)
            pltpu.sync_copy(cnt_local.at[pl.ds(0, ROWS_PER_TILE)],
                            cnt_out.at[pl.ds(r0, ROWS_PER_TILE)])

        pltpu.sync_copy(acc_sh.at[pl.ds(r0, ROWS_PER_TILE)],
                        acc_out.at[cid, pl.ds(r0, ROWS_PER_TILE)])

    return k(h2, srcp, dstp, zrows, zcnt)


def _tc_apply(h, acc, cnt, wt, b2):
    R = 1000

    def body(h_ref, acc_ref, cnt_ref, wt_ref, b_ref, o_ref):
        hb = h_ref[...]
        deg = jnp.maximum(cnt_ref[...], 1.0)
        c0 = acc_ref[0] / deg
        c1 = acc_ref[1] / deg
        z = (
            jnp.dot(hb, wt_ref[0:D, :], preferred_element_type=jnp.float32)
            + jnp.dot(c0, wt_ref[D:D + HD, :], preferred_element_type=jnp.float32)
            + jnp.dot(c1, wt_ref[D + HD:2 * D, :],
                      preferred_element_type=jnp.float32)
            + b_ref[...]
        )
        n = jnp.sqrt(jnp.sum(z * z, axis=1, keepdims=True))
        z = z / jnp.maximum(n, 1e-12)
        o_ref[...] = hb + jnp.maximum(z, 0.0)

    return pl.pallas_call(
        body,
        grid=(N_NODES // R,),
        in_specs=[
            pl.BlockSpec((R, D), lambda i: (i, 0)),
            pl.BlockSpec((NC, R, HD), lambda i: (0, i, 0)),
            pl.BlockSpec((R, 1), lambda i: (i, 0)),
            pl.BlockSpec((2 * D, D), lambda i: (0, 0)),
            pl.BlockSpec((1, D), lambda i: (0, 0)),
        ],
        out_specs=pl.BlockSpec((R, D), lambda i: (i, 0)),
        out_shape=jax.ShapeDtypeStruct((N_NODES, D), jnp.float32),
    )(h, acc, cnt, wt, b2)


@jax.jit
def kernel(h, edge_index, W, b):
    ei = edge_index.astype(jnp.int32)
    srcp = jnp.pad(ei[0].reshape(NS, EPT_RAW), ((0, 0), (0, EPT - EPT_RAW))
                   ).reshape(NS, NB, EB)
    padv = N_NODES + (jnp.arange(EPT - EPT_RAW) % (NP - N_NODES))
    padv = jnp.broadcast_to(padv[None, :], (NS, EPT - EPT_RAW)).astype(jnp.int32)
    dstp = jnp.concatenate([ei[1].reshape(NS, EPT_RAW), padv],
                           axis=1).reshape(NS, NB, EB)
    h2 = h.reshape(N_NODES, NC, HD).transpose(1, 0, 2)
    zrows = jnp.zeros((ROWS_PER_TILE, HD), jnp.float32)
    zcnt = jnp.zeros((NP,), jnp.float32)
    acc, cnt = _sc_segment_sum(h2, srcp, dstp, zrows, zcnt)
    wt = W.T
    b2 = b.reshape(1, D)
    return _tc_apply(h, acc, cnt.reshape(NP, 1), wt, b2)


# speedup vs baseline: 1.6956x; 1.0398x over previous
import functools

import jax
import jax.numpy as jnp
from jax import lax
from jax.experimental import pallas as pl
from jax.experimental.pallas import tpu as pltpu
from jax.experimental.pallas import tpu_sc as plsc

N_NODES = 10000
N_EDGES = 320000
D = 128
HD = D // 2

NC = 2
NS = 16
EDGES_PER_TILE = N_EDGES // NS
EB = 125
NB = EDGES_PER_TILE // EB
NP = 10240
ROWS_PER_TILE = NP // NS
CW = 16


def _sc_segment_sum(h2, src3, dst3, zrows, zcnt, ones1):
    mesh = plsc.VectorSubcoreMesh(
        core_axis_name="c", subcore_axis_name="s", num_cores=NC, num_subcores=NS
    )

    @functools.partial(
        pl.kernel,
        mesh=mesh,
        compiler_params=pltpu.CompilerParams(use_tc_tiling_on_sc=False),
        out_type=[
            jax.ShapeDtypeStruct((NC, NP, HD), jnp.float32),
            jax.ShapeDtypeStruct((NC, NP, CW), jnp.float32),
        ],
        scratch_types=[
            pltpu.VMEM((NB, EB), jnp.int32),
            pltpu.VMEM((NB, EB), jnp.int32),
            pltpu.VMEM((3, EB, HD), jnp.float32),
            pltpu.VMEM((EB, CW), jnp.float32),
            pltpu.SemaphoreType.DMA,
            pltpu.SemaphoreType.DMA,
            pltpu.SemaphoreType.DMA,
            pltpu.VMEM_SHARED((NP, HD), jnp.float32),
            pltpu.VMEM_SHARED((NP, CW), jnp.float32),
        ],
    )
    def k(h_hbm, src_hbm, dst_hbm, zrows_hbm, zcnt_hbm, ones_hbm,
          acc_out, cnt_out, src_v, dst_v, rows_v, ones_v, sem_g, sem_s, sem_c,
          acc_sh, cnt_sh):
        cid = lax.axis_index("c")
        sid = lax.axis_index("s")

        pltpu.sync_copy(src_hbm.at[sid], src_v)
        pltpu.sync_copy(dst_hbm.at[sid], dst_v)
        pltpu.sync_copy(ones_hbm, ones_v)

        r0 = sid * ROWS_PER_TILE
        pltpu.sync_copy(zrows_hbm, acc_sh.at[pl.ds(r0, ROWS_PER_TILE)])
        pltpu.sync_copy(zcnt_hbm, cnt_sh.at[pl.ds(r0, ROWS_PER_TILE)])
        plsc.subcore_barrier()

        hc = h_hbm.at[cid]

        pltpu.async_copy(hc.at[src_v.at[0]], rows_v.at[0], sem_g)

        def body(j, carry):
            a = lax.rem(j, 3)

            @pl.when(j >= 2)
            def _():
                pltpu.make_async_copy(zrows_hbm.at[pl.ds(0, EB)], rows_v.at[a],
                                      sem_s).wait()

            @pl.when(j < NB - 1)
            def _():
                pltpu.async_copy(hc.at[src_v.at[j + 1]],
                                 rows_v.at[lax.rem(j + 1, 3)], sem_g)

            pltpu.make_async_copy(hc.at[src_v.at[j]], rows_v.at[a],
                                  sem_g).wait()
            pltpu.async_copy(rows_v.at[a], acc_sh.at[dst_v.at[j]], sem_s,
                             add=True)

            @pl.when(lax.rem(j, 2) == cid)
            def _():
                pltpu.async_copy(ones_v, cnt_sh.at[dst_v.at[j]], sem_c,
                                 add=True)

            return carry

        lax.fori_loop(0, NB, body, 0)

        pltpu.make_async_copy(zrows_hbm.at[pl.ds(0, EB)], rows_v.at[0],
                              sem_s).wait()
        pltpu.make_async_copy(zrows_hbm.at[pl.ds(0, EB)], rows_v.at[1],
                              sem_s).wait()

        def drain_counts(j, carry):
            pltpu.make_async_copy(ones_hbm, ones_v, sem_c).wait()
            return carry

        lax.fori_loop(0, NB // 2, drain_counts, 0)
        plsc.subcore_barrier()

        pltpu.sync_copy(acc_sh.at[pl.ds(r0, ROWS_PER_TILE)],
                        acc_out.at[cid, pl.ds(r0, ROWS_PER_TILE)])
        pltpu.sync_copy(cnt_sh.at[pl.ds(r0, ROWS_PER_TILE)],
                        cnt_out.at[cid, pl.ds(r0, ROWS_PER_TILE)])

    return k(h2, src3, dst3, zrows, zcnt, ones1)


def _tc_apply(h, acc, cnt, wt, b2):
    R = 1000

    def body(h_ref, acc_ref, cnt_ref, wt_ref, b_ref, o_ref):
        hb = h_ref[...]
        deg = jnp.maximum(cnt_ref[0, :, 0:1] + cnt_ref[1, :, 0:1], 1.0)
        c0 = acc_ref[0] / deg
        c1 = acc_ref[1] / deg
        z = (
            jnp.dot(hb, wt_ref[0:D, :], preferred_element_type=jnp.float32)
            + jnp.dot(c0, wt_ref[D:D + HD, :], preferred_element_type=jnp.float32)
            + jnp.dot(c1, wt_ref[D + HD:2 * D, :],
                      preferred_element_type=jnp.float32)
            + b_ref[...]
        )
        n = jnp.sqrt(jnp.sum(z * z, axis=1, keepdims=True))
        z = z / jnp.maximum(n, 1e-12)
        o_ref[...] = hb + jnp.maximum(z, 0.0)

    return pl.pallas_call(
        body,
        grid=(N_NODES // R,),
        in_specs=[
            pl.BlockSpec((R, D), lambda i: (i, 0)),
            pl.BlockSpec((NC, R, HD), lambda i: (0, i, 0)),
            pl.BlockSpec((NC, R, CW), lambda i: (0, i, 0)),
            pl.BlockSpec((2 * D, D), lambda i: (0, 0)),
            pl.BlockSpec((1, D), lambda i: (0, 0)),
        ],
        out_specs=pl.BlockSpec((R, D), lambda i: (i, 0)),
        out_shape=jax.ShapeDtypeStruct((N_NODES, D), jnp.float32),
    )(h, acc, cnt, wt, b2)


@jax.jit
def kernel(h, edge_index, W, b):
    ei = edge_index.astype(jnp.int32)
    src3 = ei[0].reshape(NS, NB, EB)
    dst3 = ei[1].reshape(NS, NB, EB)
    h2 = jnp.stack([h[:, :HD], h[:, HD:]])
    zrows = jnp.zeros((ROWS_PER_TILE, HD), jnp.float32)
    zcnt = jnp.zeros((ROWS_PER_TILE, CW), jnp.float32)
    ones1 = jnp.ones((EB, CW), jnp.float32)
    acc, cnt = _sc_segment_sum(h2, src3, dst3, zrows, zcnt, ones1)
    wt = W.T
    b2 = b.reshape(1, D)
    return _tc_apply(h, acc, cnt, wt, b2)


# speedup vs baseline: 1.7641x; 1.0403x over previous
import functools

import jax
import jax.numpy as jnp
from jax import lax
from jax.experimental import pallas as pl
from jax.experimental.pallas import tpu as pltpu
from jax.experimental.pallas import tpu_sc as plsc

N_NODES = 10000
N_EDGES = 320000
D = 128
HD = D // 2

NC = 2
NS = 16
EDGES_PER_TILE = N_EDGES // NS
EB = 125
NB = EDGES_PER_TILE // EB
NP = 10240
ROWS_PER_TILE = NP // NS
CW = 16


def _sc_segment_sum(h2, src3, dst3, zrows, zcnt, ones1):
    mesh = plsc.VectorSubcoreMesh(
        core_axis_name="c", subcore_axis_name="s", num_cores=NC, num_subcores=NS
    )

    @functools.partial(
        pl.kernel,
        mesh=mesh,
        compiler_params=pltpu.CompilerParams(use_tc_tiling_on_sc=False),
        out_type=[
            jax.ShapeDtypeStruct((NC, NP, HD), jnp.float32),
            jax.ShapeDtypeStruct((NC, NP, CW), jnp.float32),
        ],
        scratch_types=[
            pltpu.VMEM((NB, EB), jnp.int32),
            pltpu.VMEM((NB, EB), jnp.int32),
            pltpu.VMEM((3, EB, HD), jnp.float32),
            pltpu.VMEM((EB, CW), jnp.float32),
            pltpu.SemaphoreType.DMA,
            pltpu.SemaphoreType.DMA,
            pltpu.SemaphoreType.DMA,
            pltpu.VMEM_SHARED((NP, HD), jnp.float32),
            pltpu.VMEM_SHARED((NP, CW), jnp.float32),
        ],
    )
    def k(h_hbm, src_hbm, dst_hbm, zrows_hbm, zcnt_hbm, ones_hbm,
          acc_out, cnt_out, src_v, dst_v, rows_v, ones_v, sem_g, sem_s, sem_c,
          acc_sh, cnt_sh):
        cid = lax.axis_index("c")
        sid = lax.axis_index("s")

        pltpu.sync_copy(src_hbm.at[sid], src_v)
        pltpu.sync_copy(dst_hbm.at[sid], dst_v)
        pltpu.sync_copy(ones_hbm, ones_v)

        r0 = sid * ROWS_PER_TILE
        pltpu.sync_copy(zrows_hbm, acc_sh.at[pl.ds(r0, ROWS_PER_TILE)])
        pltpu.sync_copy(zcnt_hbm, cnt_sh.at[pl.ds(r0, ROWS_PER_TILE)])
        plsc.subcore_barrier()

        hc = h_hbm.at[cid]

        pltpu.async_copy(hc.at[src_v.at[0]], rows_v.at[0], sem_g)

        def body(j, carry):
            a = lax.rem(j, 3)

            @pl.when(j >= 2)
            def _():
                pltpu.make_async_copy(zrows_hbm.at[pl.ds(0, EB)], rows_v.at[a],
                                      sem_s).wait()

            @pl.when(j < NB - 1)
            def _():
                pltpu.async_copy(hc.at[src_v.at[j + 1]],
                                 rows_v.at[lax.rem(j + 1, 3)], sem_g)

            pltpu.make_async_copy(hc.at[src_v.at[j]], rows_v.at[a],
                                  sem_g).wait()
            pltpu.async_copy(rows_v.at[a], acc_sh.at[dst_v.at[j]], sem_s,
                             add=True)

            @pl.when(lax.rem(j, 2) == cid)
            def _():
                pltpu.async_copy(ones_v, cnt_sh.at[dst_v.at[j]], sem_c,
                                 add=True)

            return carry

        lax.fori_loop(0, NB, body, 0)

        pltpu.make_async_copy(zrows_hbm.at[pl.ds(0, EB)], rows_v.at[0],
                              sem_s).wait()
        pltpu.make_async_copy(zrows_hbm.at[pl.ds(0, EB)], rows_v.at[1],
                              sem_s).wait()

        def drain_counts(j, carry):
            pltpu.make_async_copy(ones_hbm, ones_v, sem_c).wait()
            return carry

        lax.fori_loop(0, NB // 2, drain_counts, 0)
        plsc.subcore_barrier()

        pltpu.sync_copy(acc_sh.at[pl.ds(r0, ROWS_PER_TILE)],
                        acc_out.at[cid, pl.ds(r0, ROWS_PER_TILE)])
        pltpu.sync_copy(cnt_sh.at[pl.ds(r0, ROWS_PER_TILE)],
                        cnt_out.at[cid, pl.ds(r0, ROWS_PER_TILE)])

    return k(h2, src3, dst3, zrows, zcnt, ones1)


def _tc_split(h):
    R = 2000

    def body(h_ref, o_ref):
        o_ref[0] = h_ref[:, 0:HD]
        o_ref[1] = h_ref[:, HD:D]

    return pl.pallas_call(
        body,
        grid=(N_NODES // R,),
        in_specs=[pl.BlockSpec((R, D), lambda i: (i, 0))],
        out_specs=pl.BlockSpec((NC, R, HD), lambda i: (0, i, 0)),
        out_shape=jax.ShapeDtypeStruct((NC, N_NODES, HD), jnp.float32),
    )(h)


def _tc_apply(h, acc, cnt, wt, b2):
    R = 1000

    def body(h_ref, acc_ref, cnt_ref, wt_ref, b_ref, o_ref):
        hb = h_ref[...]
        deg = jnp.maximum(cnt_ref[0, :, 0:1] + cnt_ref[1, :, 0:1], 1.0)
        c0 = acc_ref[0] / deg
        c1 = acc_ref[1] / deg
        z = (
            jnp.dot(hb, wt_ref[0:D, :], preferred_element_type=jnp.float32)
            + jnp.dot(c0, wt_ref[D:D + HD, :], preferred_element_type=jnp.float32)
            + jnp.dot(c1, wt_ref[D + HD:2 * D, :],
                      preferred_element_type=jnp.float32)
            + b_ref[...]
        )
        n = jnp.sqrt(jnp.sum(z * z, axis=1, keepdims=True))
        z = z / jnp.maximum(n, 1e-12)
        o_ref[...] = hb + jnp.maximum(z, 0.0)

    return pl.pallas_call(
        body,
        grid=(N_NODES // R,),
        in_specs=[
            pl.BlockSpec((R, D), lambda i: (i, 0)),
            pl.BlockSpec((NC, R, HD), lambda i: (0, i, 0)),
            pl.BlockSpec((NC, R, CW), lambda i: (0, i, 0)),
            pl.BlockSpec((2 * D, D), lambda i: (0, 0)),
            pl.BlockSpec((1, D), lambda i: (0, 0)),
        ],
        out_specs=pl.BlockSpec((R, D), lambda i: (i, 0)),
        out_shape=jax.ShapeDtypeStruct((N_NODES, D), jnp.float32),
    )(h, acc, cnt, wt, b2)


@jax.jit
def kernel(h, edge_index, W, b):
    ei = edge_index.astype(jnp.int32)
    src3 = ei[0].reshape(NS, NB, EB)
    dst3 = ei[1].reshape(NS, NB, EB)
    h2 = _tc_split(h)
    zrows = jnp.zeros((ROWS_PER_TILE, HD), jnp.float32)
    zcnt = jnp.zeros((ROWS_PER_TILE, CW), jnp.float32)
    ones1 = jnp.ones((EB, CW), jnp.float32)
    acc, cnt = _sc_segment_sum(h2, src3, dst3, zrows, zcnt, ones1)
    wt = W.T
    b2 = b.reshape(1, D)
    return _tc_apply(h, acc, cnt, wt, b2)


# speedup vs baseline: 1.9613x; 1.1118x over previous
import functools

import jax
import jax.numpy as jnp
from jax import lax
from jax.experimental import pallas as pl
from jax.experimental.pallas import tpu as pltpu
from jax.experimental.pallas import tpu_sc as plsc

N_NODES = 10000
N_EDGES = 320000
D = 128
HD = D // 2

NC = 2
NS = 16
EDGES_PER_TILE = N_EDGES // NS
EB = 125
NB = EDGES_PER_TILE // EB
NP = 10240
ROWS_PER_TILE = NP // NS
CW = 16


def _sc_segment_sum(h2, src3, dst3, zrows, zcnt, ones1):
    mesh = plsc.VectorSubcoreMesh(
        core_axis_name="c", subcore_axis_name="s", num_cores=NC, num_subcores=NS
    )

    @functools.partial(
        pl.kernel,
        mesh=mesh,
        compiler_params=pltpu.CompilerParams(use_tc_tiling_on_sc=False),
        out_type=[
            jax.ShapeDtypeStruct((NC, NP, HD), jnp.bfloat16),
            jax.ShapeDtypeStruct((NC, NP, CW), jnp.float32),
        ],
        scratch_types=[
            pltpu.VMEM((NB, EB), jnp.int32),
            pltpu.VMEM((NB, EB), jnp.int32),
            pltpu.VMEM((3, EB, HD), jnp.bfloat16),
            pltpu.VMEM((EB, CW), jnp.float32),
            pltpu.SemaphoreType.DMA,
            pltpu.SemaphoreType.DMA,
            pltpu.SemaphoreType.DMA,
            pltpu.VMEM_SHARED((NP, HD), jnp.bfloat16),
            pltpu.VMEM_SHARED((NP, CW), jnp.float32),
        ],
    )
    def k(h_hbm, src_hbm, dst_hbm, zrows_hbm, zcnt_hbm, ones_hbm,
          acc_out, cnt_out, src_v, dst_v, rows_v, ones_v, sem_g, sem_s, sem_c,
          acc_sh, cnt_sh):
        cid = lax.axis_index("c")
        sid = lax.axis_index("s")

        pltpu.sync_copy(src_hbm.at[sid], src_v)
        pltpu.sync_copy(dst_hbm.at[sid], dst_v)
        pltpu.sync_copy(ones_hbm, ones_v)

        r0 = sid * ROWS_PER_TILE
        pltpu.sync_copy(zrows_hbm, acc_sh.at[pl.ds(r0, ROWS_PER_TILE)])
        pltpu.sync_copy(zcnt_hbm, cnt_sh.at[pl.ds(r0, ROWS_PER_TILE)])
        plsc.subcore_barrier()

        hc = h_hbm.at[cid]

        pltpu.async_copy(hc.at[src_v.at[0]], rows_v.at[0], sem_g)

        def body(j, carry):
            a = lax.rem(j, 3)

            @pl.when(j >= 2)
            def _():
                pltpu.make_async_copy(zrows_hbm.at[pl.ds(0, EB)], rows_v.at[a],
                                      sem_s).wait()

            @pl.when(j < NB - 1)
            def _():
                pltpu.async_copy(hc.at[src_v.at[j + 1]],
                                 rows_v.at[lax.rem(j + 1, 3)], sem_g)

            pltpu.make_async_copy(hc.at[src_v.at[j]], rows_v.at[a],
                                  sem_g).wait()
            pltpu.async_copy(rows_v.at[a], acc_sh.at[dst_v.at[j]], sem_s,
                             add=True)

            @pl.when(lax.rem(j, 2) == cid)
            def _():
                pltpu.async_copy(ones_v, cnt_sh.at[dst_v.at[j]], sem_c,
                                 add=True)

            return carry

        lax.fori_loop(0, NB, body, 0)

        pltpu.make_async_copy(zrows_hbm.at[pl.ds(0, EB)], rows_v.at[0],
                              sem_s).wait()
        pltpu.make_async_copy(zrows_hbm.at[pl.ds(0, EB)], rows_v.at[1],
                              sem_s).wait()

        def drain_counts(j, carry):
            pltpu.make_async_copy(ones_hbm, ones_v, sem_c).wait()
            return carry

        lax.fori_loop(0, NB // 2, drain_counts, 0)
        plsc.subcore_barrier()

        pltpu.sync_copy(acc_sh.at[pl.ds(r0, ROWS_PER_TILE)],
                        acc_out.at[cid, pl.ds(r0, ROWS_PER_TILE)])
        pltpu.sync_copy(cnt_sh.at[pl.ds(r0, ROWS_PER_TILE)],
                        cnt_out.at[cid, pl.ds(r0, ROWS_PER_TILE)])

    return k(h2, src3, dst3, zrows, zcnt, ones1)


def _tc_split(h):
    R = 2000

    def body(h_ref, o_ref):
        o_ref[0] = h_ref[:, 0:HD].astype(jnp.bfloat16)
        o_ref[1] = h_ref[:, HD:D].astype(jnp.bfloat16)

    return pl.pallas_call(
        body,
        grid=(N_NODES // R,),
        in_specs=[pl.BlockSpec((R, D), lambda i: (i, 0))],
        out_specs=pl.BlockSpec((NC, R, HD), lambda i: (0, i, 0)),
        out_shape=jax.ShapeDtypeStruct((NC, N_NODES, HD), jnp.bfloat16),
    )(h)


def _tc_apply(h, acc, cnt, wt, b2):
    R = 1000

    def body(h_ref, acc_ref, cnt_ref, wt_ref, b_ref, o_ref):
        hb = h_ref[...]
        deg = jnp.maximum(cnt_ref[0, :, 0:1] + cnt_ref[1, :, 0:1], 1.0)
        c0 = acc_ref[0].astype(jnp.float32) / deg
        c1 = acc_ref[1].astype(jnp.float32) / deg
        z = (
            jnp.dot(hb, wt_ref[0:D, :], preferred_element_type=jnp.float32)
            + jnp.dot(c0, wt_ref[D:D + HD, :], preferred_element_type=jnp.float32)
            + jnp.dot(c1, wt_ref[D + HD:2 * D, :],
                      preferred_element_type=jnp.float32)
            + b_ref[...]
        )
        n = jnp.sqrt(jnp.sum(z * z, axis=1, keepdims=True))
        z = z / jnp.maximum(n, 1e-12)
        o_ref[...] = hb + jnp.maximum(z, 0.0)

    return pl.pallas_call(
        body,
        grid=(N_NODES // R,),
        in_specs=[
            pl.BlockSpec((R, D), lambda i: (i, 0)),
            pl.BlockSpec((NC, R, HD), lambda i: (0, i, 0)),
            pl.BlockSpec((NC, R, CW), lambda i: (0, i, 0)),
            pl.BlockSpec((2 * D, D), lambda i: (0, 0)),
            pl.BlockSpec((1, D), lambda i: (0, 0)),
        ],
        out_specs=pl.BlockSpec((R, D), lambda i: (i, 0)),
        out_shape=jax.ShapeDtypeStruct((N_NODES, D), jnp.float32),
    )(h, acc, cnt, wt, b2)


@jax.jit
def kernel(h, edge_index, W, b):
    ei = edge_index.astype(jnp.int32)
    src3 = ei[0].reshape(NS, NB, EB)
    dst3 = ei[1].reshape(NS, NB, EB)
    h2 = _tc_split(h)
    zrows = jnp.zeros((ROWS_PER_TILE, HD), jnp.bfloat16)
    zcnt = jnp.zeros((ROWS_PER_TILE, CW), jnp.float32)
    ones1 = jnp.ones((EB, CW), jnp.float32)
    acc, cnt = _sc_segment_sum(h2, src3, dst3, zrows, zcnt, ones1)
    wt = W.T
    b2 = b.reshape(1, D)
    return _tc_apply(h, acc, cnt, wt, b2)
